# Initial kernel scaffold; baseline (speedup 1.0000x reference)
import jax
import jax.numpy as jnp
from jax.experimental import pallas as pl


def kernel(hidden_states, wq, wk, wv, wo, ln1, ln2, wg, wu, wd, tn_norm, tn_wg, tn_wu, tn_wd, router_w, router_b):
    raise NotImplementedError("write your pallas kernel here")


# speedup vs baseline: 1.1693x; 1.1693x over previous
import jax
import jax.numpy as jnp
from jax.experimental import pallas as pl
from jax.experimental.pallas import tpu as pltpu

T = 2048
D = 2048
NH = 16
DH = 128
FF = 8192
FFT = 2048
EPS = 1e-6
CAPK = 1024
SCALE = 1.0 / (DH ** 0.5)

BQ = 256
FT = 256
NT = 256


def _bf16(x):
    return x.astype(jnp.bfloat16)


def _rmsnorm_bf16(x, w):
    v = jnp.mean(x * x, axis=-1, keepdims=True)
    return _bf16(x * jax.lax.rsqrt(v + EPS) * w)


def _chunked_rmsnorm(src_ref, ln_ref, dst_ref, chunk):
    def body(mb, c):
        rows = pl.ds(mb * chunk, chunk)
        dst_ref[rows, :] = _rmsnorm_bf16(src_ref[rows, :], ln_ref[...])
        return c
    jax.lax.fori_loop(0, T // chunk, body, 0)


def _dot(a, b):
    return jax.lax.dot_general(a, b, (((1,), (0,)), ((), ())),
                               preferred_element_type=jnp.float32)


def _qkv_kernel(x_ref, ln_ref, wq_ref, wk_ref, wv_ref,
                q_ref, k_ref, v_ref, xn_ref):
    @pl.when(pl.program_id(0) == 0)
    def _():
        _chunked_rmsnorm(x_ref, ln_ref, xn_ref, 256)
    xn = xn_ref[...]
    q_ref[...] = _bf16(_dot(xn, _bf16(wq_ref[...])))
    k_ref[...] = _bf16(_dot(xn, _bf16(wk_ref[...])))
    v_ref[...] = _bf16(_dot(xn, _bf16(wv_ref[...])))


def _qkv(x, ln1, wq, wk, wv):
    grid = (D // NT,)
    wspec = pl.BlockSpec((D, NT), lambda i: (0, i))
    ospec = pl.BlockSpec((T, NT), lambda i: (0, i))
    return pl.pallas_call(
        _qkv_kernel,
        grid=grid,
        in_specs=[
            pl.BlockSpec((T, D), lambda i: (0, 0)),
            pl.BlockSpec((1, D), lambda i: (0, 0)),
            wspec, wspec, wspec,
        ],
        out_specs=[ospec, ospec, ospec],
        out_shape=[jax.ShapeDtypeStruct((T, D), jnp.bfloat16)] * 3,
        scratch_shapes=[pltpu.VMEM((T, D), jnp.bfloat16)],
    )(x, ln1, wq, wk, wv)


def _attn_kernel(q_ref, k_ref, v_ref, o_ref):
    qb = pl.program_id(1)
    s = jax.lax.dot_general(q_ref[...], k_ref[...],
                            (((1,), (1,)), ((), ())),
                            preferred_element_type=jnp.float32)
    s = s * SCALE
    row = qb * BQ + jax.lax.broadcasted_iota(jnp.int32, (BQ, T), 0)
    col = jax.lax.broadcasted_iota(jnp.int32, (BQ, T), 1)
    s = jnp.where(col <= row, s, -1e30)
    m = jnp.max(s, axis=-1, keepdims=True)
    p = jnp.exp(s - m)
    p = _bf16(p / jnp.sum(p, axis=-1, keepdims=True))
    o_ref[...] = _bf16(_dot(p, v_ref[...]))


def _attention(q, k, v):
    grid = (NH, T // BQ)
    kvspec = pl.BlockSpec((T, DH), lambda h, i: (0, h))
    return pl.pallas_call(
        _attn_kernel,
        grid=grid,
        in_specs=[
            pl.BlockSpec((BQ, DH), lambda h, i: (i, h)),
            kvspec, kvspec,
        ],
        out_specs=pl.BlockSpec((BQ, DH), lambda h, i: (i, h)),
        out_shape=jax.ShapeDtypeStruct((T, D), jnp.bfloat16),
    )(q, k, v)


def _oproj_kernel(a_ref, wo_ref, x_ref, h_ref):
    h_ref[...] = x_ref[...] + _dot(a_ref[...], _bf16(wo_ref[...]))


def _o_proj(attn, wo, x):
    grid = (D // NT,)
    return pl.pallas_call(
        _oproj_kernel,
        grid=grid,
        in_specs=[
            pl.BlockSpec((T, D), lambda i: (0, 0)),
            pl.BlockSpec((D, NT), lambda i: (0, i)),
            pl.BlockSpec((T, NT), lambda i: (0, i)),
        ],
        out_specs=pl.BlockSpec((T, NT), lambda i: (0, i)),
        out_shape=jax.ShapeDtypeStruct((T, D), jnp.float32),
    )(attn, wo, x)


MB = 256


def _mlp_kernel(h_ref, ln_ref, wg_ref, wu_ref, wd_ref, out_ref, xn_ref,
                *, residual):
    i = pl.program_id(0)

    @pl.when(i == 0)
    def _():
        def initbody(mb, c):
            rows = pl.ds(mb * MB, MB)
            h = h_ref[rows, :]
            v = jnp.mean(h * h, axis=-1, keepdims=True)
            xn_ref[rows, :] = _bf16(h * jax.lax.rsqrt(v + EPS) * ln_ref[...])
            if residual:
                out_ref[rows, :] = h
            else:
                out_ref[rows, :] = jnp.zeros((MB, D), jnp.float32)
            return c
        jax.lax.fori_loop(0, T // MB, initbody, 0)

    wg = _bf16(wg_ref[...])
    wu = _bf16(wu_ref[...])
    wd = _bf16(wd_ref[...])

    def body(mb, carry):
        rows = pl.ds(mb * MB, MB)
        xn = xn_ref[rows, :]
        g = _dot(xn, wg)
        u = _dot(xn, wu)
        a = _bf16(jax.nn.silu(g) * u)
        out_ref[rows, :] += _dot(a, wd)
        return carry

    jax.lax.fori_loop(0, T // MB, body, 0)


def _mlp(h, ln, wg, wu, wd, ff, residual):
    import functools
    grid = (ff // FT,)
    return pl.pallas_call(
        functools.partial(_mlp_kernel, residual=residual),
        grid=grid,
        in_specs=[
            pl.BlockSpec((T, D), lambda i: (0, 0)),
            pl.BlockSpec((1, D), lambda i: (0, 0)),
            pl.BlockSpec((D, FT), lambda i: (0, i)),
            pl.BlockSpec((D, FT), lambda i: (0, i)),
            pl.BlockSpec((FT, D), lambda i: (i, 0)),
        ],
        out_specs=pl.BlockSpec((T, D), lambda i: (0, 0)),
        out_shape=jax.ShapeDtypeStruct((T, D), jnp.float32),
        scratch_shapes=[pltpu.VMEM((T, D), jnp.bfloat16)],
    )(h, ln, wg, wu, wd)


LCH = 256
NLCH = T // LCH
HI0 = 0x3F800000


def _loss_kernel(proc_ref, h0_ref, pred_ref, w_ref, b_ref, out_ref,
                 g_scr, l_scr, tpn_scr, sp_scr):
    i = pl.program_id(0)
    proc = proc_ref[...]
    h0 = h0_ref[...]
    act = proc - h0
    diff = pred_ref[...] - act
    dst = jnp.sum(act * act, axis=-1, keepdims=True) * (1.0 / D)
    dch = jnp.sum(diff * diff, axis=-1, keepdims=True) * (1.0 / D)
    g = jax.nn.sigmoid(dst - dch)
    l = jnp.sum(h0 * w_ref[...], axis=-1, keepdims=True) + b_ref[...]
    sp = jnp.maximum(l, 0.0) + jnp.log1p(jnp.exp(-jnp.abs(l)))

    g_scr[pl.ds(i * LCH, LCH), :] = g
    l_scr[pl.ds(i * LCH, LCH), :] = l

    @pl.when(i == 0)
    def _():
        tpn_scr[...] = jnp.zeros_like(tpn_scr)
        sp_scr[...] = jnp.zeros_like(sp_scr)

    tpn_scr[...] += jnp.sum(dch, axis=0, keepdims=True)
    sp_scr[...] += jnp.sum(sp, axis=0, keepdims=True)

    @pl.when(i == NLCH - 1)
    def _():
        gv = g_scr[...]
        lv = l_scr[...]
        key = jax.lax.bitcast_convert_type(gv, jnp.int32)

        def body(_, carry):
            lo, hi = carry
            mid = lo + (hi - lo + 1) // 2
            cnt = jnp.sum(jnp.where(key >= mid, 1.0, 0.0),
                          axis=0, keepdims=True)
            cond = cnt >= CAPK
            return jnp.where(cond, mid, lo), jnp.where(cond, hi, mid - 1)

        lo0 = jnp.zeros((1, 1), jnp.int32)
        hi0 = jnp.full((1, 1), HI0, jnp.int32)
        tstar, _ = jax.lax.fori_loop(0, 31, body, (lo0, hi0))

        gt = key > tstar
        eq = key == tstar
        ngt = jnp.sum(jnp.where(gt, 1.0, 0.0), axis=0, keepdims=True)
        m = CAPK - ngt
        idx = jax.lax.broadcasted_iota(jnp.int32, (T, 1), 0)

        def body2(_, carry):
            lo, hi = carry
            mid = (lo + hi) // 2
            cnt = jnp.sum(jnp.where(eq & (idx < mid), 1.0, 0.0),
                          axis=0, keepdims=True)
            cond = cnt >= m
            return jnp.where(cond, lo, mid + 1), jnp.where(cond, mid, hi)

        lo2 = jnp.zeros((1, 1), jnp.int32)
        hi2 = jnp.full((1, 1), T, jnp.int32)
        jstar, _ = jax.lax.fori_loop(0, 12, body2, (lo2, hi2))

        sel = gt | (eq & (idx < jstar))
        topk_sum = jnp.sum(jnp.where(sel, lv, 0.0), axis=0, keepdims=True)
        tpn = tpn_scr[...] * (1.0 / T)
        causal = (sp_scr[...] - topk_sum) * (1.0 / T)
        out_ref[...] = 0.05 * tpn + 0.01 * causal


def _loss(proc, h0, pred, w_row, b):
    grid = (NLCH,)
    rspec = pl.BlockSpec((LCH, D), lambda i: (i, 0))
    return pl.pallas_call(
        _loss_kernel,
        grid=grid,
        in_specs=[
            rspec, rspec, rspec,
            pl.BlockSpec((1, D), lambda i: (0, 0)),
            pl.BlockSpec((1, 1), lambda i: (0, 0)),
        ],
        out_specs=pl.BlockSpec((1, 1), lambda i: (0, 0)),
        out_shape=jax.ShapeDtypeStruct((1, 1), jnp.float32),
        scratch_shapes=[
            pltpu.VMEM((T, 1), jnp.float32),
            pltpu.VMEM((T, 1), jnp.float32),
            pltpu.VMEM((1, 1), jnp.float32),
            pltpu.VMEM((1, 1), jnp.float32),
        ],
    )(proc, h0, pred, w_row, b)


def kernel(hidden_states, wq, wk, wv, wo, ln1, ln2, wg, wu, wd,
           tn_norm, tn_wg, tn_wu, tn_wd, router_w, router_b):
    x = hidden_states.reshape(T, D)
    q, k, v = _qkv(x, ln1.reshape(1, D), wq, wk, wv)
    attn = _attention(q, k, v)
    h = _o_proj(attn, wo, x)
    processed = _mlp(h, ln2.reshape(1, D), wg, wu, wd, FF, residual=True)
    prev = jnp.concatenate(
        [jnp.zeros((1, D), jnp.float32), processed[:-1]], axis=0)
    pred = _mlp(prev, tn_norm.reshape(1, D), tn_wg, tn_wu, tn_wd, FFT,
                residual=False)
    aux = _loss(processed, x, pred, router_w.reshape(1, D),
                router_b.reshape(1, 1))
    return processed.reshape(1, T, D), aux.reshape(())
